# baseline (speedup 1.0000x reference)
import jax
import jax.numpy as jnp
import numpy as np
from jax import lax
from jax.experimental import pallas as pl
from jax.experimental.pallas import tpu as pltpu
from jax.experimental.pallas import tpu_sc as plsc

OUTPUT_SIZE = 768
WIN = 16
NUM_BINS = 96

NC = 2
NS = 16
NW = NC * NS
L = 16

B_TOTAL = 4 * 8192
B_PER_W = B_TOTAL // NW
N_WINDOWS = B_PER_W // WIN
_bins = (440.0 * 2.0 ** ((np.arange(NUM_BINS, dtype=np.float32) - 48.0) / 12.0)
         ).astype(np.float32)
_MIDS = tuple(float(m) for m in
              ((_bins[:-1] + _bins[1:]) * 0.5).astype(np.float32))


def _pitch_encode_body(sig_hbm, table_hbm, out_hbm,
                       sig_v, table_v, table_sh, sem0):
    sid = lax.axis_index("s")
    wid = sid * NC + lax.axis_index("c")
    base = wid * B_PER_W

    rows_per_stager = NUM_BINS // 4

    @pl.when(sid < 4)
    def _stage_table():
        pltpu.sync_copy(
            table_hbm.at[pl.ds(sid * rows_per_stager, rows_per_stager)],
            table_sh.at[pl.ds(sid * rows_per_stager, rows_per_stager)])

    pltpu.sync_copy(sig_hbm.at[pl.ds(base, B_PER_W)], sig_v)

    plsc.subcore_barrier()
    pltpu.sync_copy(table_sh, table_v)

    iota = lax.iota(jnp.int32, L)
    dnums = lax.GatherDimensionNumbers(
        offset_dims=(), collapsed_slice_dims=(0,), start_index_map=(0,))

    def lane_perm(x, idx):
        return lax.gather(x, idx[:, None], dnums, slice_sizes=(1,),
                          mode=lax.GatherScatterMode.PROMISE_IN_BOUNDS)

    perms = [iota ^ sh for sh in (1, 2, 4, 8)]

    def window_dma_wait():
        pltpu.make_async_copy(table_v.at[pl.ds(0, WIN)],
                              out_hbm.at[pl.ds(base, WIN)], sem0).wait()

    def emit_window(w):
        v = sig_v[pl.ds(w * WIN, WIN)]
        s = v
        for p in perms:
            s = s + lane_perm(s, p)
        sig = jnp.where(v != 0.0, s * (1.0 / WIN),
                        jnp.zeros((L,), jnp.float32))
        acc = jnp.zeros((L,), jnp.int32)
        one = jnp.ones((L,), jnp.int32)
        zero = jnp.zeros((L,), jnp.int32)
        for m in _MIDS:
            acc = acc + jnp.where(sig > m, one, zero)
        for j in range(WIN):
            pltpu.async_copy(table_v.at[pl.ds(acc[j], 1)],
                             out_hbm.at[pl.ds(base + w * WIN + j, 1)], sem0)

    LAG = 8

    def window_body(w, carry):
        emit_window(w)

        @pl.when(w >= LAG)
        def _drain_lagged():
            window_dma_wait()

        return carry

    lax.fori_loop(0, N_WINDOWS, window_body, 0)
    for _ in range(LAG):
        window_dma_wait()


@jax.jit
def _pitch_encode(signals_flat, emb_table):
    mesh = plsc.VectorSubcoreMesh(core_axis_name="c", subcore_axis_name="s")
    return pl.kernel(
        _pitch_encode_body,
        out_type=jax.ShapeDtypeStruct((B_TOTAL, OUTPUT_SIZE), jnp.float32),
        mesh=mesh,
        scratch_types=[
            pltpu.VMEM((B_PER_W,), jnp.float32),
            pltpu.VMEM((NUM_BINS, OUTPUT_SIZE), jnp.float32),
            pltpu.VMEM_SHARED((NUM_BINS, OUTPUT_SIZE), jnp.float32),
            pltpu.SemaphoreType.DMA,
        ],
    )(signals_flat, emb_table)


def kernel(signals, emb_table):
    if signals.ndim == 3 and signals.shape[-1] == 1:
        signals = signals[..., 0]
    B, W = signals.shape
    out = _pitch_encode(signals.reshape(-1), emb_table)
    return out.reshape(B, W, OUTPUT_SIZE)


# speedup vs baseline: 1.0026x; 1.0026x over previous
import jax
import jax.numpy as jnp
import numpy as np
from jax import lax
from jax.experimental import pallas as pl
from jax.experimental.pallas import tpu as pltpu
from jax.experimental.pallas import tpu_sc as plsc

OUTPUT_SIZE = 768
WIN = 16
NUM_BINS = 96

NC = 2
NS = 16
NW = NC * NS
L = 16

B_TOTAL = 4 * 8192
B_PER_W = B_TOTAL // NW
N_WINDOWS = B_PER_W // WIN
_bins = (440.0 * 2.0 ** ((np.arange(NUM_BINS, dtype=np.float32) - 48.0) / 12.0)
         ).astype(np.float32)
_MIDS = tuple(float(m) for m in
              ((_bins[:-1] + _bins[1:]) * 0.5).astype(np.float32))


def _pitch_encode_body(sig_hbm, table_hbm, out_hbm,
                       sig_v, table_v, table_sh, sem0):
    sid = lax.axis_index("s")
    wid = sid * NC + lax.axis_index("c")
    base = wid * B_PER_W

    @pl.when(sid == 0)
    def _stage_table():
        pltpu.sync_copy(table_hbm, table_sh)

    pltpu.sync_copy(sig_hbm.at[pl.ds(base, B_PER_W)], sig_v)

    plsc.subcore_barrier()
    pltpu.sync_copy(table_sh, table_v)

    iota = lax.iota(jnp.int32, L)
    dnums = lax.GatherDimensionNumbers(
        offset_dims=(), collapsed_slice_dims=(0,), start_index_map=(0,))

    def lane_perm(x, idx):
        return lax.gather(x, idx[:, None], dnums, slice_sizes=(1,),
                          mode=lax.GatherScatterMode.PROMISE_IN_BOUNDS)

    perms = [iota ^ sh for sh in (1, 2, 4, 8)]

    def window_dma_wait():
        pltpu.make_async_copy(table_v.at[pl.ds(0, WIN)],
                              out_hbm.at[pl.ds(base, WIN)], sem0).wait()

    def emit_window(w):
        v = sig_v[pl.ds(w * WIN, WIN)]
        s = v
        for p in perms:
            s = s + lane_perm(s, p)
        sig = jnp.where(v != 0.0, s * (1.0 / WIN),
                        jnp.zeros((L,), jnp.float32))
        acc = jnp.zeros((L,), jnp.int32)
        one = jnp.ones((L,), jnp.int32)
        zero = jnp.zeros((L,), jnp.int32)
        for m in _MIDS:
            acc = acc + jnp.where(sig > m, one, zero)
        for j in range(WIN):
            pltpu.async_copy(table_v.at[pl.ds(acc[j], 1)],
                             out_hbm.at[pl.ds(base + w * WIN + j, 1)], sem0)

    LAG = 8

    def window_body(w, carry):
        emit_window(w)

        @pl.when(w >= LAG)
        def _drain_lagged():
            window_dma_wait()

        return carry

    lax.fori_loop(0, N_WINDOWS, window_body, 0)
    for _ in range(LAG):
        window_dma_wait()


@jax.jit
def _pitch_encode(signals_flat, emb_table):
    mesh = plsc.VectorSubcoreMesh(core_axis_name="c", subcore_axis_name="s")
    return pl.kernel(
        _pitch_encode_body,
        out_type=jax.ShapeDtypeStruct((B_TOTAL, OUTPUT_SIZE), jnp.float32),
        mesh=mesh,
        scratch_types=[
            pltpu.VMEM((B_PER_W,), jnp.float32),
            pltpu.VMEM((NUM_BINS, OUTPUT_SIZE), jnp.float32),
            pltpu.VMEM_SHARED((NUM_BINS, OUTPUT_SIZE), jnp.float32),
            pltpu.SemaphoreType.DMA,
        ],
    )(signals_flat, emb_table)


def kernel(signals, emb_table):
    if signals.ndim == 3 and signals.shape[-1] == 1:
        signals = signals[..., 0]
    B, W = signals.shape
    out = _pitch_encode(signals.reshape(-1), emb_table)
    return out.reshape(B, W, OUTPUT_SIZE)
